# Initial kernel scaffold; baseline (speedup 1.0000x reference)
import jax
import jax.numpy as jnp
from jax.experimental import pallas as pl


def kernel(positions, adj, W0, b0, W1, b1, W2, b2, W3, b3, W4, b4, W5, b5, W6, b6, W7, b7, W8, b8, W9, b9, W10, b10, W11, b11, W12, b12, W13, b13, W14, b14, W15, b15, Wr, br):
    raise NotImplementedError("write your pallas kernel here")


# speedup vs baseline: 1.3976x; 1.3976x over previous
import jax
import jax.numpy as jnp
from jax.experimental import pallas as pl

_DIMS = [(3, 60), (60, 60), (60, 60), (60, 60), (60, 120), (120, 120),
         (120, 120), (120, 150), (150, 200), (200, 210), (210, 250),
         (250, 300), (300, 300), (300, 300), (300, 300), (300, 300)]
_JOINT = 512


def _elu(x):
    return jnp.where(x > 0, x, jnp.exp(jnp.minimum(x, 0.0)) - 1.0)


def _encoder_body(*refs):
    pos_ref, adj_ref = refs[0], refs[1]
    wrefs = refs[2:-1]
    out_ref = refs[-1]

    adj = adj_ref[0]
    inv_norm = 1.0 / jnp.sum(adj, axis=1, keepdims=True)

    x = pos_ref[0]
    for i, (fi, fo) in enumerate(_DIMS):
        w = wrefs[2 * i][...]
        b = wrefs[2 * i + 1][...]
        support = jnp.dot(x, w, preferred_element_type=jnp.float32)
        s = max(fo // 3, 2)
        c = min(fo, 128)
        col_c = jax.lax.broadcasted_iota(jnp.int32, (1, c), 1)
        pre = jnp.where(col_c < s, support[:, :c] * inv_norm, 0.0)
        side = jnp.dot(adj, pre, preferred_element_type=jnp.float32)
        if fo > c:
            side = jnp.concatenate(
                [side, jnp.zeros((side.shape[0], fo - c), side.dtype)], axis=1)
        col_f = jax.lax.broadcasted_iota(jnp.int32, (1, fo), 1)
        out = jnp.where(col_f < s, side, support) + b
        x = _elu(out)

    wr = wrefs[-2][...]
    br = wrefs[-1][...]
    support = jnp.dot(x, wr, preferred_element_type=jnp.float32)
    out = jnp.dot(adj, support, preferred_element_type=jnp.float32) + br
    latent = jnp.max(out, axis=0, keepdims=True)
    out_ref[...] = _elu(latent).reshape(1, 1, _JOINT)


def kernel(positions, adj,
           W0, b0, W1, b1, W2, b2, W3, b3,
           W4, b4, W5, b5, W6, b6, W7, b7,
           W8, b8, W9, b9, W10, b10, W11, b11,
           W12, b12, W13, b13, W14, b14, W15, b15,
           Wr, br):
    B, N, _ = positions.shape
    ws = [W0, b0, W1, b1, W2, b2, W3, b3, W4, b4, W5, b5, W6, b6, W7, b7,
          W8, b8, W9, b9, W10, b10, W11, b11, W12, b12, W13, b13, W14, b14,
          W15, b15]

    args = [positions, adj]
    in_specs = [
        pl.BlockSpec((1, N, 3), lambda b: (b, 0, 0)),
        pl.BlockSpec((1, N, N), lambda b: (b, 0, 0)),
    ]
    for i, (fi, fo) in enumerate(_DIMS):
        args.append(ws[2 * i])
        in_specs.append(pl.BlockSpec((fi, fo), lambda b: (0, 0)))
        args.append(ws[2 * i + 1].reshape(1, fo))
        in_specs.append(pl.BlockSpec((1, fo), lambda b: (0, 0)))
    args.append(Wr)
    in_specs.append(pl.BlockSpec(Wr.shape, lambda b: (0, 0)))
    args.append(br.reshape(1, _JOINT))
    in_specs.append(pl.BlockSpec((1, _JOINT), lambda b: (0, 0)))

    out = pl.pallas_call(
        _encoder_body,
        grid=(B,),
        in_specs=in_specs,
        out_specs=pl.BlockSpec((1, 1, _JOINT), lambda b: (b, 0, 0)),
        out_shape=jax.ShapeDtypeStruct((B, 1, _JOINT), jnp.float32),
    )(*args)
    return out.reshape(B, _JOINT)
